# Initial kernel scaffold; baseline (speedup 1.0000x reference)
import jax
import jax.numpy as jnp
from jax.experimental import pallas as pl


def kernel(x, w0, w1, w2, w3, w4, a_h, a_w, a_c, b_h, b_w, b_c):
    raise NotImplementedError("write your pallas kernel here")


# speedup vs baseline: 1.6955x; 1.6955x over previous
import functools

import jax
import jax.numpy as jnp
from jax import lax
from jax.experimental import pallas as pl
from jax.experimental.pallas import tpu as pltpu
from jax.experimental.pallas import tpu_sc as plsc

B, C, H, W = 32, 3, 224, 224
K = 16
S = 16
NSTRIP = 14
STRIP_WORDS = C * 16 * W
NNODE = 31
NC = 2

_TBL = [
    [0, 0, 0, 0], [0, 0, 0, 1], [0, 1, 0, -1], [0, 1, 0, 0],
    [0, 0, 1, -1], [0, 0, 1, 0], [0, 1, 1, -2], [0, 1, 1, -1],
    [1, -1, -1, 1], [1, -1, -1, 2], [1, 0, -1, 0], [1, 0, -1, 1],
    [1, -1, 0, 0], [1, -1, 0, 1], [1, 0, 0, -1], [1, 0, 0, 0],
]


def _prep_body(w_ref, t_ref, ac_ref, ah_ref, aw_ref, bc_ref, bh_ref, bw_ref,
               coeff_ref, idx_ref):
    w = w_ref[...]
    m = jnp.max(w, axis=-1, keepdims=True)
    e = jnp.exp(w - m)
    p = e / jnp.sum(e, axis=-1, keepdims=True)
    t = t_ref[...]
    c = lax.dot_general(p.reshape(NNODE * K, 16), t,
                        (((1,), (0,)), ((), ())),
                        preferred_element_type=jnp.float32)
    coeff_ref[...] = jnp.broadcast_to(
        c.reshape(NNODE, K, 4)[..., None], (NNODE, K, 4, 16))

    ia = ac_ref[...] * (16 * W) + ah_ref[...] * W + aw_ref[...]
    ib = bc_ref[...] * (16 * W) + bh_ref[...] * W + bw_ref[...]
    base = jnp.stack([ia, ib])[..., None]
    lane = lax.broadcasted_iota(jnp.int32, (2, K, S, 16), 3) * 16
    idx_ref[...] = jnp.minimum(base + lane, STRIP_WORDS - 1)


_prep = pl.pallas_call(
    _prep_body,
    out_shape=(
        jax.ShapeDtypeStruct((NNODE, K, 4, 16), jnp.float32),
        jax.ShapeDtypeStruct((2, K, S, 16), jnp.int32),
    ),
)


def _sc_body(x1, coeffh, idxh, out_hbm, strip_v, coeff_v, idx_v, outb_v):
    b = lax.axis_index("s") * NC + lax.axis_index("c")
    pltpu.sync_copy(coeffh, coeff_v)
    pltpu.sync_copy(idxh, idx_v)

    def strip_body(st, carry):
        for c in range(C):
            pltpu.sync_copy(
                x1.at[pl.ds((b * C + c) * (H * W) + st * 3584, 3584)],
                strip_v.at[pl.ds(c * 3584, 3584)])

        def k_body(k, carry2):
            def comb(av, bv, node):
                base = (node * K + k) * 64
                c0 = coeff_v[pl.ds(base, 16)]
                c1 = coeff_v[pl.ds(base + 16, 16)]
                c2 = coeff_v[pl.ds(base + 32, 16)]
                c3 = coeff_v[pl.ds(base + 48, 16)]
                return (c0 + c1 * av) + bv * (c2 + c3 * av)

            vals = []
            for s in range(S):
                iav = idx_v[pl.ds((k * S + s) * 16, 16)]
                ibv = idx_v[pl.ds((K * S + k * S + s) * 16, 16)]
                av = plsc.load_gather(strip_v, [iav])
                bv = plsc.load_gather(strip_v, [ibv])
                vals.append(comb(av, bv, s))
            node = S
            while len(vals) > 1:
                vals = [comb(vals[2 * g], vals[2 * g + 1], node + g)
                        for g in range(len(vals) // 2)]
                node += len(vals)
            outb_v[pl.ds(k * 224 + st * 16, 16)] = vals[0]
            return carry2

        lax.fori_loop(0, K, k_body, 0)
        return carry

    lax.fori_loop(0, NSTRIP, strip_body, 0)
    pltpu.sync_copy(outb_v, out_hbm.at[pl.ds(b * (K * 224), K * 224)])


_sc_main = functools.partial(
    pl.kernel,
    mesh=plsc.VectorSubcoreMesh(core_axis_name="c", subcore_axis_name="s"),
    compiler_params=pltpu.CompilerParams(needs_layout_passes=False),
    out_type=jax.ShapeDtypeStruct((B * K * NSTRIP * 16,), jnp.float32),
    scratch_types=[
        pltpu.VMEM((STRIP_WORDS,), jnp.float32),
        pltpu.VMEM((NNODE * K * 4 * 16,), jnp.float32),
        pltpu.VMEM((2 * K * S * 16,), jnp.int32),
        pltpu.VMEM((K * NSTRIP * 16,), jnp.float32),
    ],
)(_sc_body)


def kernel(x, w0, w1, w2, w3, w4, a_h, a_w, a_c, b_h, b_w, b_c):
    wcat = jnp.concatenate([w0, w1, w2, w3, w4], axis=0)
    tbl = jnp.asarray(_TBL, dtype=jnp.float32)
    coeff, idx = _prep(wcat, tbl, a_c[:, 0], a_h[:, 0], a_w[:, 0],
                       b_c[:, 0], b_h[:, 0], b_w[:, 0])
    out = _sc_main(x.reshape(-1), coeff.reshape(-1), idx.reshape(-1))
    return out.reshape(B, K, NSTRIP, 16)[..., :14].reshape(B, K, 14 * NSTRIP, 1)


# speedup vs baseline: 2.1604x; 1.2742x over previous
import functools

import jax
import jax.numpy as jnp
from jax import lax
from jax.experimental import pallas as pl
from jax.experimental.pallas import tpu as pltpu
from jax.experimental.pallas import tpu_sc as plsc

B, C, H, W = 32, 3, 224, 224
K = 16
S = 16
P = 196
NSTRIP = 14
NNODE = 31
NC = 2

_TBL = [
    [0, 0, 0, 0], [0, 0, 0, 1], [0, 1, 0, -1], [0, 1, 0, 0],
    [0, 0, 1, -1], [0, 0, 1, 0], [0, 1, 1, -2], [0, 1, 1, -1],
    [1, -1, -1, 1], [1, -1, -1, 2], [1, 0, -1, 0], [1, 0, -1, 1],
    [1, -1, 0, 0], [1, -1, 0, 1], [1, 0, 0, -1], [1, 0, 0, 0],
]


def _prep_body(w0, w1, w2, w3, w4, t_ref,
               ac_ref, ah_ref, aw_ref, bc_ref, bh_ref, bw_ref,
               coeff_ref, row_ref, col_ref):
    w = jnp.concatenate(
        [w0[...], w1[...], w2[...], w3[...], w4[...]], axis=0)
    m = jnp.max(w, axis=-1, keepdims=True)
    e = jnp.exp(w - m)
    p = e / jnp.sum(e, axis=-1, keepdims=True)
    t = t_ref[...]
    c = lax.dot_general(p.reshape(NNODE * K, 16), t,
                        (((1,), (0,)), ((), ())),
                        preferred_element_type=jnp.float32)
    coeff_ref[...] = jnp.broadcast_to(
        c.reshape(NNODE, K, 4)[..., None], (NNODE, K, 4, 16))

    rows = jnp.stack([ac_ref[:, 0, :] * 16 + ah_ref[:, 0, :],
                      bc_ref[:, 0, :] * 16 + bh_ref[:, 0, :]])
    cols = jnp.stack([aw_ref[:, 0, :], bw_ref[:, 0, :]])
    row_ref[...] = jnp.broadcast_to(rows[..., None], (2, K, S, 16))
    lane = lax.broadcasted_iota(jnp.int32, (2, K, S, 16), 3) * 16
    col_ref[...] = jnp.minimum(cols[..., None] + lane, W - 1)


_idx_spec = pl.BlockSpec((K, 8, S), lambda i: (0, 0, 0))
_prep = pl.pallas_call(
    _prep_body,
    grid=(1,),
    in_specs=[
        pl.BlockSpec((16, K, 16), lambda i: (0, 0, 0)),
        pl.BlockSpec((8, K, 16), lambda i: (0, 0, 0)),
        pl.BlockSpec((4, K, 16), lambda i: (0, 0, 0)),
        pl.BlockSpec((2, K, 16), lambda i: (0, 0, 0)),
        pl.BlockSpec((1, K, 16), lambda i: (0, 0, 0)),
        pl.BlockSpec((16, 4), lambda i: (0, 0)),
        _idx_spec, _idx_spec, _idx_spec, _idx_spec, _idx_spec, _idx_spec,
    ],
    out_specs=(
        pl.BlockSpec((NNODE, K, 4, 16), lambda i: (0, 0, 0, 0)),
        pl.BlockSpec((2, K, S, 16), lambda i: (0, 0, 0, 0)),
        pl.BlockSpec((2, K, S, 16), lambda i: (0, 0, 0, 0)),
    ),
    out_shape=(
        jax.ShapeDtypeStruct((NNODE, K, 4, 16), jnp.float32),
        jax.ShapeDtypeStruct((2, K, S, 16), jnp.int32),
        jax.ShapeDtypeStruct((2, K, S, 16), jnp.int32),
    ),
)


def _sc_body(x, coeffh, rowh, colh, out_hbm, strip_v, coeff_v, row_v, col_v,
             outb_v):
    b = lax.axis_index("s") * NC + lax.axis_index("c")
    pltpu.sync_copy(coeffh, coeff_v)
    pltpu.sync_copy(rowh, row_v)
    pltpu.sync_copy(colh, col_v)

    def strip_body(st, carry):
        for c in range(C):
            pltpu.sync_copy(x.at[b, c, pl.ds(st * 16, 16), :],
                            strip_v.at[pl.ds(c * 16, 16), :])

        def k_body(k, carry2):
            def comb(av, bv, node):
                base = (node * K + k) * 64
                c0 = coeff_v[pl.ds(base, 16)]
                c1 = coeff_v[pl.ds(base + 16, 16)]
                c2 = coeff_v[pl.ds(base + 32, 16)]
                c3 = coeff_v[pl.ds(base + 48, 16)]
                return (c0 + c1 * av) + bv * (c2 + c3 * av)

            vals = []
            for s in range(S):
                ra = row_v[pl.ds((k * S + s) * 16, 16)]
                ca = col_v[pl.ds((k * S + s) * 16, 16)]
                rb = row_v[pl.ds((K * S + k * S + s) * 16, 16)]
                cb = col_v[pl.ds((K * S + k * S + s) * 16, 16)]
                av = plsc.load_gather(strip_v, [ra, ca])
                bv = plsc.load_gather(strip_v, [rb, cb])
                vals.append(comb(av, bv, s))
            node = S
            while len(vals) > 1:
                vals = [comb(vals[2 * g], vals[2 * g + 1], node + g)
                        for g in range(len(vals) // 2)]
                node += len(vals)
            outb_v[pl.ds(k * 224 + st * 16, 16)] = vals[0]
            return carry2

        lax.fori_loop(0, K, k_body, 0)
        return carry

    lax.fori_loop(0, NSTRIP, strip_body, 0)
    pltpu.sync_copy(outb_v, out_hbm.at[pl.ds(b * (K * 224), K * 224)])


_sc_main = functools.partial(
    pl.kernel,
    mesh=plsc.VectorSubcoreMesh(core_axis_name="c", subcore_axis_name="s"),
    compiler_params=pltpu.CompilerParams(needs_layout_passes=False),
    out_type=jax.ShapeDtypeStruct((B * K * NSTRIP * 16,), jnp.float32),
    scratch_types=[
        pltpu.VMEM((C * 16, W), jnp.float32),
        pltpu.VMEM((NNODE * K * 4 * 16,), jnp.float32),
        pltpu.VMEM((2 * K * S * 16,), jnp.int32),
        pltpu.VMEM((2 * K * S * 16,), jnp.int32),
        pltpu.VMEM((K * NSTRIP * 16,), jnp.float32),
    ],
)(_sc_body)


def kernel(x, w0, w1, w2, w3, w4, a_h, a_w, a_c, b_h, b_w, b_c):
    tbl = jnp.asarray(_TBL, dtype=jnp.float32)
    coeff, rowt, colt = _prep(w0, w1, w2, w3, w4, tbl,
                              a_c, a_h, a_w, b_c, b_h, b_w)
    out = _sc_main(x, coeff.reshape(-1), rowt.reshape(-1), colt.reshape(-1))
    return out.reshape(B, K, NSTRIP, 16)[..., :14].reshape(B, K, P, 1)


# speedup vs baseline: 3.1763x; 1.4703x over previous
import functools

import jax
import jax.numpy as jnp
from jax import lax
from jax.experimental import pallas as pl
from jax.experimental.pallas import tpu as pltpu
from jax.experimental.pallas import tpu_sc as plsc

B, C, H, W = 32, 3, 224, 224
K = 16
S = 16
P = 196
NSTRIP = 14
NNODE = 31
NC = 2

_TBL = [
    [0, 0, 0, 0], [0, 0, 0, 1], [0, 1, 0, -1], [0, 1, 0, 0],
    [0, 0, 1, -1], [0, 0, 1, 0], [0, 1, 1, -2], [0, 1, 1, -1],
    [1, -1, -1, 1], [1, -1, -1, 2], [1, 0, -1, 0], [1, 0, -1, 1],
    [1, -1, 0, 0], [1, -1, 0, 1], [1, 0, 0, -1], [1, 0, 0, 0],
]


def _prep_body(w0, w1, w2, w3, w4, t_ref,
               ac_ref, ah_ref, aw_ref, bc_ref, bh_ref, bw_ref,
               coeff_ref, row_ref, col_ref):
    w = jnp.concatenate(
        [w0[...], w1[...], w2[...], w3[...], w4[...]], axis=0)
    m = jnp.max(w, axis=-1, keepdims=True)
    e = jnp.exp(w - m)
    p = e / jnp.sum(e, axis=-1, keepdims=True)
    t = t_ref[...]
    c = lax.dot_general(p.reshape(NNODE * K, 16), t,
                        (((1,), (0,)), ((), ())),
                        preferred_element_type=jnp.float32)
    coeff_ref[...] = jnp.broadcast_to(
        c.reshape(NNODE, K, 4)[..., None], (NNODE, K, 4, 16))

    rows = jnp.stack([ac_ref[:, 0, :] * 16 + ah_ref[:, 0, :],
                      bc_ref[:, 0, :] * 16 + bh_ref[:, 0, :]])
    cols = jnp.stack([aw_ref[:, 0, :], bw_ref[:, 0, :]])
    row_ref[...] = jnp.broadcast_to(rows[..., None], (2, K, S, 16))
    lane = lax.broadcasted_iota(jnp.int32, (2, K, S, 16), 3) * 16
    col_ref[...] = jnp.minimum(cols[..., None] + lane, W - 1)


_idx_spec = pl.BlockSpec((K, 8, S), lambda i: (0, 0, 0))
_prep = pl.pallas_call(
    _prep_body,
    grid=(1,),
    in_specs=[
        pl.BlockSpec((16, K, 16), lambda i: (0, 0, 0)),
        pl.BlockSpec((8, K, 16), lambda i: (0, 0, 0)),
        pl.BlockSpec((4, K, 16), lambda i: (0, 0, 0)),
        pl.BlockSpec((2, K, 16), lambda i: (0, 0, 0)),
        pl.BlockSpec((1, K, 16), lambda i: (0, 0, 0)),
        pl.BlockSpec((16, 4), lambda i: (0, 0)),
        _idx_spec, _idx_spec, _idx_spec, _idx_spec, _idx_spec, _idx_spec,
    ],
    out_specs=(
        pl.BlockSpec((NNODE, K, 4, 16), lambda i: (0, 0, 0, 0)),
        pl.BlockSpec((2, K, S, 16), lambda i: (0, 0, 0, 0)),
        pl.BlockSpec((2, K, S, 16), lambda i: (0, 0, 0, 0)),
    ),
    out_shape=(
        jax.ShapeDtypeStruct((NNODE, K, 4, 16), jnp.float32),
        jax.ShapeDtypeStruct((2, K, S, 16), jnp.int32),
        jax.ShapeDtypeStruct((2, K, S, 16), jnp.int32),
    ),
)


NPAIR = NSTRIP // 2


def _sc_body(x, coeffh, rowh, colh, out_hbm, strip0_v, strip1_v,
             coeff_v, row_v, col_v, outb_v, sem0, sem1):
    b = lax.axis_index("s") * NC + lax.axis_index("c")

    def issue(pair, buf, sem):
        handles = []
        for j in range(2):
            st = 2 * pair + j
            for c in range(C):
                handles.append(pltpu.async_copy(
                    x.at[b, c, pl.ds(st * 16, 16), :],
                    buf.at[pl.ds(j * 48 + c * 16, 16), :], sem))
        return handles

    pend = issue(0, strip0_v, sem0)
    pltpu.sync_copy(coeffh, coeff_v)
    pltpu.sync_copy(rowh, row_v)
    pltpu.sync_copy(colh, col_v)

    for pair in range(NPAIR):
        buf = strip0_v if pair % 2 == 0 else strip1_v
        if pair + 1 < NPAIR:
            nxt = issue(pair + 1,
                        strip1_v if pair % 2 == 0 else strip0_v,
                        sem1 if pair % 2 == 0 else sem0)
        else:
            nxt = None
        for h in pend:
            h.wait()
        pend = nxt

        def k_body(k, carry2, _buf=buf, _pair=pair):
            def comb(av, bv, node):
                base = (node * K + k) * 64
                c0 = coeff_v[pl.ds(base, 16)]
                c1 = coeff_v[pl.ds(base + 16, 16)]
                c2 = coeff_v[pl.ds(base + 32, 16)]
                c3 = coeff_v[pl.ds(base + 48, 16)]
                return ((c0 + c1 * av[0]) + bv[0] * (c2 + c3 * av[0]),
                        (c0 + c1 * av[1]) + bv[1] * (c2 + c3 * av[1]))

            vals = []
            for s in range(S):
                ra = row_v[pl.ds((k * S + s) * 16, 16)]
                ca = col_v[pl.ds((k * S + s) * 16, 16)]
                rb = row_v[pl.ds((K * S + k * S + s) * 16, 16)]
                cb = col_v[pl.ds((K * S + k * S + s) * 16, 16)]
                av = (plsc.load_gather(_buf, [ra, ca]),
                      plsc.load_gather(_buf, [ra + 48, ca]))
                bv = (plsc.load_gather(_buf, [rb, cb]),
                      plsc.load_gather(_buf, [rb + 48, cb]))
                vals.append(comb(av, bv, s))
            node = S
            while len(vals) > 1:
                vals = [comb(vals[2 * g], vals[2 * g + 1], node + g)
                        for g in range(len(vals) // 2)]
                node += len(vals)
            outb_v[pl.ds(k * 224 + (2 * _pair) * 16, 16)] = vals[0][0]
            outb_v[pl.ds(k * 224 + (2 * _pair + 1) * 16, 16)] = vals[0][1]
            return carry2

        lax.fori_loop(0, K, k_body, 0)

    pltpu.sync_copy(outb_v, out_hbm.at[pl.ds(b * (K * 224), K * 224)])


_sc_main = functools.partial(
    pl.kernel,
    mesh=plsc.VectorSubcoreMesh(core_axis_name="c", subcore_axis_name="s"),
    compiler_params=pltpu.CompilerParams(needs_layout_passes=False),
    out_type=jax.ShapeDtypeStruct((B * K * NSTRIP * 16,), jnp.float32),
    scratch_types=[
        pltpu.VMEM((2 * C * 16, W), jnp.float32),
        pltpu.VMEM((2 * C * 16, W), jnp.float32),
        pltpu.VMEM((NNODE * K * 4 * 16,), jnp.float32),
        pltpu.VMEM((2 * K * S * 16,), jnp.int32),
        pltpu.VMEM((2 * K * S * 16,), jnp.int32),
        pltpu.VMEM((K * NSTRIP * 16,), jnp.float32),
        pltpu.SemaphoreType.DMA,
        pltpu.SemaphoreType.DMA,
    ],
)(_sc_body)


def kernel(x, w0, w1, w2, w3, w4, a_h, a_w, a_c, b_h, b_w, b_c):
    tbl = jnp.asarray(_TBL, dtype=jnp.float32)
    coeff, rowt, colt = _prep(w0, w1, w2, w3, w4, tbl,
                              a_c, a_h, a_w, b_c, b_h, b_w)
    out = _sc_main(x, coeff.reshape(-1), rowt.reshape(-1), colt.reshape(-1))
    return out.reshape(B, K, NSTRIP, 16)[..., :14].reshape(B, K, P, 1)


# speedup vs baseline: 3.2282x; 1.0163x over previous
import functools

import jax
import jax.numpy as jnp
from jax import lax
from jax.experimental import pallas as pl
from jax.experimental.pallas import tpu as pltpu
from jax.experimental.pallas import tpu_sc as plsc

B, C, H, W = 32, 3, 224, 224
K = 16
S = 16
P = 196
NSTRIP = 14
NNODE = 31
NC = 2

_TBL = [
    [0, 0, 0, 0], [0, 0, 0, 1], [0, 1, 0, -1], [0, 1, 0, 0],
    [0, 0, 1, -1], [0, 0, 1, 0], [0, 1, 1, -2], [0, 1, 1, -1],
    [1, -1, -1, 1], [1, -1, -1, 2], [1, 0, -1, 0], [1, 0, -1, 1],
    [1, -1, 0, 0], [1, -1, 0, 1], [1, 0, 0, -1], [1, 0, 0, 0],
]


def _prep_body(w0, w1, w2, w3, w4, t_ref,
               ac_ref, ah_ref, aw_ref, bc_ref, bh_ref, bw_ref,
               coeff_ref, row_ref, col_ref):
    w = jnp.concatenate(
        [w0[...], w1[...], w2[...], w3[...], w4[...]], axis=0)
    m = jnp.max(w, axis=-1, keepdims=True)
    e = jnp.exp(w - m)
    p = e / jnp.sum(e, axis=-1, keepdims=True)
    t = t_ref[...]
    c = lax.dot_general(p.reshape(NNODE * K, 16), t,
                        (((1,), (0,)), ((), ())),
                        preferred_element_type=jnp.float32)
    coeff_ref[...] = jnp.broadcast_to(
        c.reshape(NNODE, K, 4)[..., None], (NNODE, K, 4, 16))

    rows = jnp.stack([ac_ref[:, 0, :] * 16 + ah_ref[:, 0, :],
                      bc_ref[:, 0, :] * 16 + bh_ref[:, 0, :]])
    cols = jnp.stack([aw_ref[:, 0, :], bw_ref[:, 0, :]])
    row_ref[...] = jnp.broadcast_to(rows[..., None], (2, K, S, 16))
    lane = lax.broadcasted_iota(jnp.int32, (2, K, S, 16), 3) * 16
    col_ref[...] = jnp.minimum(cols[..., None] + lane, W - 1)


_idx_spec = pl.BlockSpec((K, 8, S), lambda i: (0, 0, 0))
_prep = pl.pallas_call(
    _prep_body,
    grid=(1,),
    in_specs=[
        pl.BlockSpec((16, K, 16), lambda i: (0, 0, 0)),
        pl.BlockSpec((8, K, 16), lambda i: (0, 0, 0)),
        pl.BlockSpec((4, K, 16), lambda i: (0, 0, 0)),
        pl.BlockSpec((2, K, 16), lambda i: (0, 0, 0)),
        pl.BlockSpec((1, K, 16), lambda i: (0, 0, 0)),
        pl.BlockSpec((16, 4), lambda i: (0, 0)),
        _idx_spec, _idx_spec, _idx_spec, _idx_spec, _idx_spec, _idx_spec,
    ],
    out_specs=(
        pl.BlockSpec((NNODE, K, 4, 16), lambda i: (0, 0, 0, 0)),
        pl.BlockSpec((2, K, S, 16), lambda i: (0, 0, 0, 0)),
        pl.BlockSpec((2, K, S, 16), lambda i: (0, 0, 0, 0)),
    ),
    out_shape=(
        jax.ShapeDtypeStruct((NNODE, K, 4, 16), jnp.float32),
        jax.ShapeDtypeStruct((2, K, S, 16), jnp.int32),
        jax.ShapeDtypeStruct((2, K, S, 16), jnp.int32),
    ),
)


NPAIR = NSTRIP // 2


def _sc_body(x, coeffh, rowh, colh, out_hbm, strip0_v, strip1_v,
             coeff_v, row_v, col_v, outb_v, sem0, sem1):
    b = lax.axis_index("s") * NC + lax.axis_index("c")

    def issue(pair, buf, sem):
        handles = []
        for j in range(2):
            st = 2 * pair + j
            for c in range(C):
                handles.append(pltpu.async_copy(
                    x.at[b, c, pl.ds(st * 16, 16), :],
                    buf.at[pl.ds(j * 48 + c * 16, 16), :], sem))
        return handles

    pend = issue(0, strip0_v, sem0)
    pltpu.sync_copy(coeffh, coeff_v)
    pltpu.sync_copy(rowh, row_v)
    pltpu.sync_copy(colh, col_v)

    for pair in range(NPAIR):
        buf = strip0_v if pair % 2 == 0 else strip1_v
        if pair + 1 < NPAIR:
            nxt = issue(pair + 1,
                        strip1_v if pair % 2 == 0 else strip0_v,
                        sem1 if pair % 2 == 0 else sem0)
        else:
            nxt = None
        for h in pend:
            h.wait()
        pend = nxt

        def k_body(k, carry2, _buf=buf, _pair=pair):
            def comb(av, bv, node):
                base = (node * K + k) * 64
                c0 = coeff_v[pl.ds(base, 16)]
                c1 = coeff_v[pl.ds(base + 16, 16)]
                c2 = coeff_v[pl.ds(base + 32, 16)]
                c3 = coeff_v[pl.ds(base + 48, 16)]
                return ((c0 + c1 * av[0]) + bv[0] * (c2 + c3 * av[0]),
                        (c0 + c1 * av[1]) + bv[1] * (c2 + c3 * av[1]))

            node_off = [0, 16, 24, 28, 30]
            stack = []
            for s in range(S):
                ra = row_v[pl.ds((k * S + s) * 16, 16)]
                ca = col_v[pl.ds((k * S + s) * 16, 16)]
                rb = row_v[pl.ds((K * S + k * S + s) * 16, 16)]
                cb = col_v[pl.ds((K * S + k * S + s) * 16, 16)]
                av = (plsc.load_gather(_buf, [ra, ca]),
                      plsc.load_gather(_buf, [ra + 48, ca]))
                bv = (plsc.load_gather(_buf, [rb, cb]),
                      plsc.load_gather(_buf, [rb + 48, cb]))
                cur, lvl, g = comb(av, bv, s), 1, s
                while g % 2 == 1:
                    cur = comb(stack.pop(), cur, node_off[lvl] + g // 2)
                    lvl, g = lvl + 1, g // 2
                stack.append(cur)
            root = stack.pop()
            assert not stack
            outb_v[pl.ds(k * 224 + (2 * _pair) * 16, 16)] = root[0]
            outb_v[pl.ds(k * 224 + (2 * _pair + 1) * 16, 16)] = root[1]
            return carry2

        lax.fori_loop(0, K, k_body, 0)

    pltpu.sync_copy(outb_v, out_hbm.at[pl.ds(b * (K * 224), K * 224)])


_sc_main = functools.partial(
    pl.kernel,
    mesh=plsc.VectorSubcoreMesh(core_axis_name="c", subcore_axis_name="s"),
    compiler_params=pltpu.CompilerParams(needs_layout_passes=False),
    out_type=jax.ShapeDtypeStruct((B * K * NSTRIP * 16,), jnp.float32),
    scratch_types=[
        pltpu.VMEM((2 * C * 16, W), jnp.float32),
        pltpu.VMEM((2 * C * 16, W), jnp.float32),
        pltpu.VMEM((NNODE * K * 4 * 16,), jnp.float32),
        pltpu.VMEM((2 * K * S * 16,), jnp.int32),
        pltpu.VMEM((2 * K * S * 16,), jnp.int32),
        pltpu.VMEM((K * NSTRIP * 16,), jnp.float32),
        pltpu.SemaphoreType.DMA,
        pltpu.SemaphoreType.DMA,
    ],
)(_sc_body)


def kernel(x, w0, w1, w2, w3, w4, a_h, a_w, a_c, b_h, b_w, b_c):
    tbl = jnp.asarray(_TBL, dtype=jnp.float32)
    coeff, rowt, colt = _prep(w0, w1, w2, w3, w4, tbl,
                              a_c, a_h, a_w, b_c, b_h, b_w)
    out = _sc_main(x, coeff.reshape(-1), rowt.reshape(-1), colt.reshape(-1))
    return out.reshape(B, K, NSTRIP, 16)[..., :14].reshape(B, K, P, 1)


# speedup vs baseline: 3.3837x; 1.0482x over previous
import functools

import jax
import jax.numpy as jnp
from jax import lax
from jax.experimental import pallas as pl
from jax.experimental.pallas import tpu as pltpu
from jax.experimental.pallas import tpu_sc as plsc

B, C, H, W = 32, 3, 224, 224
K = 16
S = 16
P = 196
NSTRIP = 14
NNODE = 31
NC = 2

_TBL = [
    [0, 0, 0, 0], [0, 0, 0, 1], [0, 1, 0, -1], [0, 1, 0, 0],
    [0, 0, 1, -1], [0, 0, 1, 0], [0, 1, 1, -2], [0, 1, 1, -1],
    [1, -1, -1, 1], [1, -1, -1, 2], [1, 0, -1, 0], [1, 0, -1, 1],
    [1, -1, 0, 0], [1, -1, 0, 1], [1, 0, 0, -1], [1, 0, 0, 0],
]


def _make_m():
    import numpy as np
    t = np.asarray(_TBL, dtype=np.float32)
    m = np.zeros((32, 128), dtype=np.float32)
    for q in range(8):
        half, j = q // 4, q % 4
        for u in range(16):
            m[half * 16 + u, q * 16:(q + 1) * 16] = t[u, j]
    return m


def _prep_body(w_ref, m_ref, coeff_ref):
    w = w_ref[...]
    h1, h2 = w[:, :16], w[:, 16:]

    def sm(h):
        e = jnp.exp(h - jnp.max(h, axis=-1, keepdims=True))
        return e / jnp.sum(e, axis=-1, keepdims=True)

    p = jnp.concatenate([sm(h1), sm(h2)], axis=1)
    coeff_ref[...] = lax.dot_general(p, m_ref[...],
                                     (((1,), (0,)), ((), ())),
                                     preferred_element_type=jnp.float32)


_M = _make_m()

_prep = pl.pallas_call(
    _prep_body,
    out_shape=jax.ShapeDtypeStruct((NNODE * K // 2, 128), jnp.float32),
)


NPAIR = NSTRIP // 2


def _sc_body(x, coeffh, rowh, colh, out_hbm, strip0_v, strip1_v,
             coeff_v, row_v, col_v, outb_v, sem0, sem1):
    b = lax.axis_index("s") * NC + lax.axis_index("c")

    def issue(pair, buf, sem):
        handles = []
        for j in range(2):
            st = 2 * pair + j
            for c in range(C):
                handles.append(pltpu.async_copy(
                    x.at[b, c, pl.ds(st * 16, 16), :],
                    buf.at[pl.ds(j * 48 + c * 16, 16), :], sem))
        return handles

    pend = issue(0, strip0_v, sem0)
    pltpu.sync_copy(coeffh, coeff_v)
    pltpu.sync_copy(rowh, row_v)
    pltpu.sync_copy(colh, col_v)

    for pair in range(NPAIR):
        buf = strip0_v if pair % 2 == 0 else strip1_v
        if pair + 1 < NPAIR:
            nxt = issue(pair + 1,
                        strip1_v if pair % 2 == 0 else strip0_v,
                        sem1 if pair % 2 == 0 else sem0)
        else:
            nxt = None
        for h in pend:
            h.wait()
        pend = nxt

        def k_body(k, carry2, _buf=buf, _pair=pair):
            def comb(av, bv, node):
                base = (node * K + k) * 64
                c0 = coeff_v[pl.ds(base, 16)]
                c1 = coeff_v[pl.ds(base + 16, 16)]
                c2 = coeff_v[pl.ds(base + 32, 16)]
                c3 = coeff_v[pl.ds(base + 48, 16)]
                return ((c0 + c1 * av[0]) + bv[0] * (c2 + c3 * av[0]),
                        (c0 + c1 * av[1]) + bv[1] * (c2 + c3 * av[1]))

            node_off = [0, 16, 24, 28, 30]
            stack = []
            for s in range(S):
                ra = row_v[pl.ds((k * S + s) * 16, 16)]
                ca = col_v[pl.ds((k * S + s) * 16, 16)]
                rb = row_v[pl.ds((K * S + k * S + s) * 16, 16)]
                cb = col_v[pl.ds((K * S + k * S + s) * 16, 16)]
                av = (plsc.load_gather(_buf, [ra, ca]),
                      plsc.load_gather(_buf, [ra + 48, ca]))
                bv = (plsc.load_gather(_buf, [rb, cb]),
                      plsc.load_gather(_buf, [rb + 48, cb]))
                cur, lvl, g = comb(av, bv, s), 1, s
                while g % 2 == 1:
                    cur = comb(stack.pop(), cur, node_off[lvl] + g // 2)
                    lvl, g = lvl + 1, g // 2
                stack.append(cur)
            root = stack.pop()
            assert not stack
            outb_v[pl.ds(k * 224 + (2 * _pair) * 16, 16)] = root[0]
            outb_v[pl.ds(k * 224 + (2 * _pair + 1) * 16, 16)] = root[1]
            return carry2

        lax.fori_loop(0, K, k_body, 0)

    pltpu.sync_copy(outb_v, out_hbm.at[pl.ds(b * (K * 224), K * 224)])


_sc_main = functools.partial(
    pl.kernel,
    mesh=plsc.VectorSubcoreMesh(core_axis_name="c", subcore_axis_name="s"),
    compiler_params=pltpu.CompilerParams(needs_layout_passes=False),
    out_type=jax.ShapeDtypeStruct((B * K * NSTRIP * 16,), jnp.float32),
    scratch_types=[
        pltpu.VMEM((2 * C * 16, W), jnp.float32),
        pltpu.VMEM((2 * C * 16, W), jnp.float32),
        pltpu.VMEM((NNODE * K * 4 * 16,), jnp.float32),
        pltpu.VMEM((2 * K * S * 16,), jnp.int32),
        pltpu.VMEM((2 * K * S * 16,), jnp.int32),
        pltpu.VMEM((K * NSTRIP * 16,), jnp.float32),
        pltpu.SemaphoreType.DMA,
        pltpu.SemaphoreType.DMA,
    ],
)(_sc_body)


def kernel(x, w0, w1, w2, w3, w4, a_h, a_w, a_c, b_h, b_w, b_c):
    wflat = jnp.concatenate([w0, w1, w2, w3, w4], axis=0).reshape(248, 32)
    coeff = _prep(wflat, jnp.asarray(_M)).reshape(-1)
    rows = jnp.stack([a_c[:, 0] * 16 + a_h[:, 0],
                      b_c[:, 0] * 16 + b_h[:, 0]])
    cols = jnp.stack([a_w[:, 0], b_w[:, 0]])
    lane = jnp.arange(16, dtype=jnp.int32) * 16
    rowt = jnp.broadcast_to(rows.reshape(-1)[:, None],
                            (2 * K * S, 16)).reshape(-1)
    colt = jnp.minimum(cols.reshape(-1)[:, None] + lane[None, :],
                       W - 1).reshape(-1)
    out = _sc_main(x, coeff, rowt, colt)
    return out.reshape(B, K, NSTRIP, 16)[..., :14].reshape(B, K, P, 1)


# speedup vs baseline: 3.6672x; 1.0838x over previous
import functools

import jax
import jax.numpy as jnp
from jax import lax
from jax.experimental import pallas as pl
from jax.experimental.pallas import tpu as pltpu
from jax.experimental.pallas import tpu_sc as plsc

B, C, H, W = 32, 3, 224, 224
K = 16
S = 16
P = 196
NSTRIP = 14
NNODE = 31
NC = 2

_TBL = [
    [0, 0, 0, 0], [0, 0, 0, 1], [0, 1, 0, -1], [0, 1, 0, 0],
    [0, 0, 1, -1], [0, 0, 1, 0], [0, 1, 1, -2], [0, 1, 1, -1],
    [1, -1, -1, 1], [1, -1, -1, 2], [1, 0, -1, 0], [1, 0, -1, 1],
    [1, -1, 0, 0], [1, -1, 0, 1], [1, 0, 0, -1], [1, 0, 0, 0],
]


def _make_m():
    import numpy as np
    t = np.asarray(_TBL, dtype=np.float32)
    m = np.zeros((32, 128), dtype=np.float32)
    for q in range(8):
        half, j = q // 4, q % 4
        for u in range(16):
            m[half * 16 + u, q * 16:(q + 1) * 16] = t[u, j]
    return m


def _prep_body(w0, w1, w2, w3, w4, m_ref, coeff_ref):
    w = jnp.concatenate(
        [w0[...], w1[...], w2[...], w3[...], w4[...]], axis=0)
    e = jnp.exp(w - jnp.max(w, axis=-1, keepdims=True))
    p = e / jnp.sum(e, axis=-1, keepdims=True)
    p4 = p.reshape(NNODE, K // 2, 2, 16)
    pe = p4[:, :, 0, :].reshape(NNODE * K // 2, 16)
    po = p4[:, :, 1, :].reshape(NNODE * K // 2, 16)
    dn = (((1,), (0,)), ((), ()))
    coeff_ref[...] = (
        lax.dot_general(pe, m_ref[pl.ds(0, 16), :], dn,
                        preferred_element_type=jnp.float32)
        + lax.dot_general(po, m_ref[pl.ds(16, 16), :], dn,
                          preferred_element_type=jnp.float32))


_M = _make_m()

_prep = pl.pallas_call(
    _prep_body,
    out_shape=jax.ShapeDtypeStruct((NNODE * K // 2, 128), jnp.float32),
)


NPAIR = NSTRIP // 2


def _sc_body(x, coeffh, rowh, colh, out_hbm, strip0_v, strip1_v,
             coeff_v, row_v, col_v, outb_v, sem0, sem1):
    b = lax.axis_index("s") * NC + lax.axis_index("c")

    def issue(pair, buf, sem):
        handles = []
        for j in range(2):
            st = 2 * pair + j
            for c in range(C):
                handles.append(pltpu.async_copy(
                    x.at[b, c, pl.ds(st * 16, 16), :],
                    buf.at[pl.ds(j * 48 + c * 16, 16), :], sem))
        return handles

    pend = issue(0, strip0_v, sem0)
    pltpu.sync_copy(coeffh, coeff_v)
    pltpu.sync_copy(rowh, row_v)
    pltpu.sync_copy(colh, col_v)

    for pair in range(NPAIR):
        buf = strip0_v if pair % 2 == 0 else strip1_v
        if pair + 1 < NPAIR:
            nxt = issue(pair + 1,
                        strip1_v if pair % 2 == 0 else strip0_v,
                        sem1 if pair % 2 == 0 else sem0)
        else:
            nxt = None
        for h in pend:
            h.wait()
        pend = nxt

        def k_body(k, carry2, _buf=buf, _pair=pair):
            def comb(av, bv, node):
                base = (node * K + k) * 64
                c0 = coeff_v[pl.ds(base, 16)]
                c1 = coeff_v[pl.ds(base + 16, 16)]
                c2 = coeff_v[pl.ds(base + 32, 16)]
                c3 = coeff_v[pl.ds(base + 48, 16)]
                return ((c0 + c1 * av[0]) + bv[0] * (c2 + c3 * av[0]),
                        (c0 + c1 * av[1]) + bv[1] * (c2 + c3 * av[1]))

            node_off = [0, 16, 24, 28, 30]
            stack = []
            for s in range(S):
                ra = row_v[pl.ds((k * S + s) * 16, 16)]
                ca = col_v[pl.ds((k * S + s) * 16, 16)]
                rb = row_v[pl.ds((K * S + k * S + s) * 16, 16)]
                cb = col_v[pl.ds((K * S + k * S + s) * 16, 16)]
                av = (plsc.load_gather(_buf, [ra, ca]),
                      plsc.load_gather(_buf, [ra + 48, ca]))
                bv = (plsc.load_gather(_buf, [rb, cb]),
                      plsc.load_gather(_buf, [rb + 48, cb]))
                cur, lvl, g = comb(av, bv, s), 1, s
                while g % 2 == 1:
                    cur = comb(stack.pop(), cur, node_off[lvl] + g // 2)
                    lvl, g = lvl + 1, g // 2
                stack.append(cur)
            root = stack.pop()
            assert not stack
            mask = lax.iota(jnp.int32, 16) < 14
            plsc.store_compressed(
                outb_v.at[pl.ds(k * P + (2 * _pair) * 14, 16)], root[0], mask=mask)
            plsc.store_compressed(
                outb_v.at[pl.ds(k * P + (2 * _pair + 1) * 14, 16)], root[1],
                mask=mask)
            return carry2

        lax.fori_loop(0, K, k_body, 0, unroll=2)

    pltpu.sync_copy(outb_v.at[pl.ds(0, K * P)],
                    out_hbm.at[pl.ds(b * (K * P), K * P)])


_sc_main = functools.partial(
    pl.kernel,
    mesh=plsc.VectorSubcoreMesh(core_axis_name="c", subcore_axis_name="s"),
    compiler_params=pltpu.CompilerParams(needs_layout_passes=False),
    out_type=jax.ShapeDtypeStruct((B * K * P,), jnp.float32),
    scratch_types=[
        pltpu.VMEM((2 * C * 16, W), jnp.float32),
        pltpu.VMEM((2 * C * 16, W), jnp.float32),
        pltpu.VMEM((NNODE * K * 4 * 16,), jnp.float32),
        pltpu.VMEM((2 * K * S * 16,), jnp.int32),
        pltpu.VMEM((2 * K * S * 16,), jnp.int32),
        pltpu.VMEM((K * P + 16,), jnp.float32),
        pltpu.SemaphoreType.DMA,
        pltpu.SemaphoreType.DMA,
    ],
)(_sc_body)


def kernel(x, w0, w1, w2, w3, w4, a_h, a_w, a_c, b_h, b_w, b_c):
    coeff = _prep(w0, w1, w2, w3, w4, jnp.asarray(_M)).reshape(-1)
    rows = jnp.stack([a_c[:, 0] * 16 + a_h[:, 0],
                      b_c[:, 0] * 16 + b_h[:, 0]])
    cols = jnp.stack([a_w[:, 0], b_w[:, 0]])
    lane = jnp.arange(16, dtype=jnp.int32) * 16
    rowt = jnp.broadcast_to(rows.reshape(-1)[:, None],
                            (2 * K * S, 16)).reshape(-1)
    colt = jnp.minimum(cols.reshape(-1)[:, None] + lane[None, :],
                       W - 1).reshape(-1)
    out = _sc_main(x, coeff, rowt, colt)
    return out.reshape(B, K, P, 1)
